# baseline (speedup 1.0000x reference)
import functools

import jax
import jax.numpy as jnp
from jax import lax
from jax.experimental import pallas as pl
from jax.experimental.pallas import tpu as pltpu
from jax.experimental.pallas import tpu_sc as plsc

D = 128
EI = 320000
E = EI // 2
NW = 32
EPW = E // NW
C = 40
NCHUNK = EPW // C
NSLOT = 10
NITER = NCHUNK // NSLOT
NGRP = (EPW + 15) // 16
EPW_PAD = NGRP * 16
STAGE = 2 * EPW_PAD


def _make_kernel():
    mesh = plsc.VectorSubcoreMesh(core_axis_name="c", subcore_axis_name="s")

    row_bufs = [pltpu.VMEM((C, D), jnp.float32) for _ in range(2 * NSLOT)]
    sems = [pltpu.SemaphoreType.DMA for _ in range(3 * NSLOT)]

    @functools.partial(
        pl.kernel,
        mesh=mesh,
        compiler_params=pltpu.CompilerParams(needs_layout_passes=False),
        out_type=(
            jax.ShapeDtypeStruct((E, D), jnp.float32),
            jax.ShapeDtypeStruct((E,), jnp.int32),
        ),
        scratch_types=[
            pltpu.VMEM((STAGE,), jnp.int32),
            pltpu.VMEM((EPW_PAD,), jnp.int32),
            pltpu.VMEM((EPW_PAD,), jnp.int32),
            pltpu.VMEM((EPW_PAD,), jnp.int32),
        ] + row_bufs + sems,
    )
    def tp_pool(x_hbm, ei_hbm, batch_hbm, out_hbm, eb_hbm,
                stage_v, src_v, dst_v, eb_all, *bufs_and_sems):
        s = bufs_and_sems[0:NSLOT]
        d = bufs_and_sems[NSLOT:2 * NSLOT]
        gs = bufs_and_sems[2 * NSLOT:3 * NSLOT]
        gd = bufs_and_sems[3 * NSLOT:4 * NSLOT]
        ss = bufs_and_sems[4 * NSLOT:5 * NSLOT]
        wid = lax.axis_index("s") * 2 + lax.axis_index("c")
        base = wid * EPW

        evens = lax.iota(jnp.int32, 16) * 2
        stage_v[pl.ds(2 * EPW, STAGE - 2 * EPW)] = jnp.zeros(
            (STAGE - 2 * EPW,), jnp.int32)

        def compact(row_off, out_idx):
            pltpu.sync_copy(ei_hbm.at[pl.ds(row_off + 2 * base, 2 * EPW)],
                            stage_v.at[pl.ds(0, 2 * EPW)])

            def grp(g, carry):
                v = plsc.load_gather(stage_v, [evens + g * 32])
                out_idx[pl.ds(g * 16, 16)] = v
                return carry

            lax.fori_loop(0, NGRP, grp, 0)

        compact(0, src_v)
        compact(EI, dst_v)

        def gather_src(ci, sk, sem):
            pltpu.async_copy(x_hbm.at[src_v.at[pl.ds(ci * C, C)]], sk, sem)

        def gather_dst(ci, dk, sem):
            pltpu.async_copy(x_hbm.at[dst_v.at[pl.ds(ci * C, C)]], dk, sem)

        def wait_gather_src(sk, sem):
            pltpu.make_async_copy(x_hbm.at[src_v.at[pl.ds(0, C)]], sk,
                                  sem).wait()

        def wait_gather_dst(dk, sem):
            pltpu.make_async_copy(x_hbm.at[dst_v.at[pl.ds(0, C)]], dk,
                                  sem).wait()

        def add(sk, dk):
            @plsc.parallel_loop(0, C, 1, unroll=2)
            def row_body(i):
                for q in range(D // 16):
                    sl = pl.ds(q * 16, 16)
                    plsc.addupdate(dk.at[i, sl], sk[i, sl])

        def store(ci, dk, sem):
            pltpu.async_copy(dk, out_hbm.at[pl.ds(base + ci * C, C)], sem)

        def wait_store(dk, sem):
            pltpu.make_async_copy(dk, out_hbm.at[pl.ds(base, C)], sem).wait()

        for k in range(NSLOT):
            gather_src(k, s[k], gs[k])
            gather_dst(k, d[k], gd[k])

        pltpu.sync_copy(batch_hbm.at[pl.ds(0, 10000)],
                        stage_v.at[pl.ds(0, 10000)])

        def eb_grp(g, carry):
            idxv = src_v[pl.ds(g * 16, 16)]
            eb_all[pl.ds(g * 16, 16)] = plsc.load_gather(stage_v, [idxv])
            return carry

        lax.fori_loop(0, NGRP, eb_grp, 0)

        def step(j, carry):
            for k in range(NSLOT):
                ci = j * NSLOT + k
                wait_gather_src(s[k], gs[k])
                wait_gather_dst(d[k], gd[k])
                add(s[k], d[k])
                store(ci, d[k], ss[k])
                wait_store(d[k], ss[k])

                @pl.when(ci + NSLOT < NCHUNK)
                def _():
                    gather_src(ci + NSLOT, s[k], gs[k])
                    gather_dst(ci + NSLOT, d[k], gd[k])

            return carry

        lax.fori_loop(0, NITER, step, 0)

        for k in range(NCHUNK - NITER * NSLOT):
            ci = NITER * NSLOT + k
            wait_gather_src(s[k], gs[k])
            wait_gather_dst(d[k], gd[k])
            add(s[k], d[k])
            store(ci, d[k], ss[k])
            wait_store(d[k], ss[k])
        pltpu.sync_copy(eb_all.at[pl.ds(0, EPW)], eb_hbm.at[pl.ds(base, EPW)])

    return tp_pool


_tp_pool = _make_kernel()


def kernel(x, edge_index, batch):
    ei_flat = edge_index.astype(jnp.int32).reshape(-1)
    batch_i32 = batch.astype(jnp.int32)
    tp_features, edge_batch = _tp_pool(x, ei_flat, batch_i32)
    return tp_features, edge_batch.astype(batch.dtype)


# speedup vs baseline: 1.0296x; 1.0296x over previous
import functools

import jax
import jax.numpy as jnp
from jax import lax
from jax.experimental import pallas as pl
from jax.experimental.pallas import tpu as pltpu
from jax.experimental.pallas import tpu_sc as plsc

D = 128
EI = 320000
E = EI // 2
NW = 32
EPW = E // NW
C = 40
NCHUNK = EPW // C
NSLOT = 5
NITER = NCHUNK // NSLOT
NGRP = (EPW + 15) // 16
EPW_PAD = NGRP * 16
STAGE = 2 * EPW_PAD


def _make_kernel():
    mesh = plsc.VectorSubcoreMesh(core_axis_name="c", subcore_axis_name="s")

    row_bufs = [pltpu.VMEM((C, D), jnp.float32) for _ in range(2 * NSLOT)]
    sems = [pltpu.SemaphoreType.DMA for _ in range(3 * NSLOT)]

    @functools.partial(
        pl.kernel,
        mesh=mesh,
        compiler_params=pltpu.CompilerParams(needs_layout_passes=False),
        out_type=(
            jax.ShapeDtypeStruct((E, D), jnp.float32),
            jax.ShapeDtypeStruct((E,), jnp.int32),
        ),
        scratch_types=[
            pltpu.VMEM((STAGE,), jnp.int32),
            pltpu.VMEM((EPW_PAD,), jnp.int32),
            pltpu.VMEM((EPW_PAD,), jnp.int32),
            pltpu.VMEM((EPW_PAD,), jnp.int32),
        ] + row_bufs + sems,
    )
    def tp_pool(x_hbm, ei_hbm, batch_hbm, out_hbm, eb_hbm,
                stage_v, src_v, dst_v, eb_all, *bufs_and_sems):
        s = bufs_and_sems[0:NSLOT]
        d = bufs_and_sems[NSLOT:2 * NSLOT]
        gs = bufs_and_sems[2 * NSLOT:3 * NSLOT]
        gd = bufs_and_sems[3 * NSLOT:4 * NSLOT]
        ss = bufs_and_sems[4 * NSLOT:5 * NSLOT]
        wid = lax.axis_index("s") * 2 + lax.axis_index("c")
        base = wid * EPW

        evens = lax.iota(jnp.int32, 16) * 2
        stage_v[pl.ds(2 * EPW, STAGE - 2 * EPW)] = jnp.zeros(
            (STAGE - 2 * EPW,), jnp.int32)

        def compact(row_off, out_idx):
            pltpu.sync_copy(ei_hbm.at[pl.ds(row_off + 2 * base, 2 * EPW)],
                            stage_v.at[pl.ds(0, 2 * EPW)])

            def grp(g, carry):
                v = plsc.load_gather(stage_v, [evens + g * 32])
                out_idx[pl.ds(g * 16, 16)] = v
                return carry

            lax.fori_loop(0, NGRP, grp, 0)

        compact(0, src_v)
        compact(EI, dst_v)

        def gather_src(ci, sk, sem):
            pltpu.async_copy(x_hbm.at[src_v.at[pl.ds(ci * C, C)]], sk, sem)

        def gather_dst(ci, dk, sem):
            pltpu.async_copy(x_hbm.at[dst_v.at[pl.ds(ci * C, C)]], dk, sem)

        def wait_gather_src(sk, sem):
            pltpu.make_async_copy(x_hbm.at[src_v.at[pl.ds(0, C)]], sk,
                                  sem).wait()

        def wait_gather_dst(dk, sem):
            pltpu.make_async_copy(x_hbm.at[dst_v.at[pl.ds(0, C)]], dk,
                                  sem).wait()

        def add(sk, dk):
            @plsc.parallel_loop(0, C, 1, unroll=2)
            def row_body(i):
                for q in range(D // 16):
                    sl = pl.ds(q * 16, 16)
                    plsc.addupdate(dk.at[i, sl], sk[i, sl])

        def store(ci, dk, sem):
            pltpu.async_copy(dk, out_hbm.at[pl.ds(base + ci * C, C)], sem)

        def wait_store(dk, sem):
            pltpu.make_async_copy(dk, out_hbm.at[pl.ds(base, C)], sem).wait()

        for k in range(NSLOT):
            gather_src(k, s[k], gs[k])
            gather_dst(k, d[k], gd[k])

        pltpu.sync_copy(batch_hbm.at[pl.ds(0, 10000)],
                        stage_v.at[pl.ds(0, 10000)])

        def eb_grp(g, carry):
            idxv = src_v[pl.ds(g * 16, 16)]
            eb_all[pl.ds(g * 16, 16)] = plsc.load_gather(stage_v, [idxv])
            return carry

        lax.fori_loop(0, NGRP, eb_grp, 0)

        def step(j, carry):
            for k in range(NSLOT):
                ci = j * NSLOT + k
                wait_gather_src(s[k], gs[k])
                wait_gather_dst(d[k], gd[k])
                add(s[k], d[k])
                store(ci, d[k], ss[k])
                wait_store(d[k], ss[k])

                @pl.when(j < NITER - 1)
                def _():
                    gather_src(ci + NSLOT, s[k], gs[k])
                    gather_dst(ci + NSLOT, d[k], gd[k])

            return carry

        lax.fori_loop(0, NITER, step, 0)
        pltpu.sync_copy(eb_all.at[pl.ds(0, EPW)], eb_hbm.at[pl.ds(base, EPW)])

    return tp_pool


_tp_pool = _make_kernel()


def kernel(x, edge_index, batch):
    ei_flat = edge_index.astype(jnp.int32).reshape(-1)
    batch_i32 = batch.astype(jnp.int32)
    tp_features, edge_batch = _tp_pool(x, ei_flat, batch_i32)
    return tp_features, edge_batch.astype(batch.dtype)
